# baseline (speedup 1.0000x reference)
import functools

import jax
import jax.numpy as jnp
from jax import lax
from jax.experimental import pallas as pl
from jax.experimental.pallas import tpu as pltpu
from jax.experimental.pallas import tpu_sc as plsc

_M = 1000000
_INV = 1.0 / 1000000.0

_NC, _NS, _L = 2, 16, 16
_NW = _NC * _NS
_N = 3276800
_PER_W = _N // _NW
_CHUNK = 12800
_NCHUNK = _PER_W // _CHUNK


def _mod_body(vals_hbm, keys_hbm, out_v_hbm, out_k_hbm,
              in0, in1, ou0, ou1, ls0, ls1, ss0, ss1):
    wid = lax.axis_index("s") * _NC + lax.axis_index("c")
    base = wid * _PER_W

    inbufs = (in0, in1)
    outbufs = (ou0, ou1)
    lsems = (ls0, ls1)
    ssems = (ss0, ss1)

    stages = []
    for c in range(_NCHUNK):
        stages.append((vals_hbm, out_v_hbm, c * _CHUNK))
    for c in range(_NCHUNK):
        stages.append((keys_hbm, out_k_hbm, c * _CHUNK))
    n_stages = len(stages)

    def start_load(i):
        src, _, off = stages[i]
        return pltpu.async_copy(
            src.at[pl.ds(base + off, _CHUNK)], inbufs[i % 2], lsems[i % 2])

    def start_store(i):
        _, dst, off = stages[i]
        return pltpu.async_copy(
            outbufs[i % 2], dst.at[pl.ds(base + off, _CHUNK)], ssems[i % 2])

    loads = {0: start_load(0), 1: start_load(1)}
    stores = {}
    for i in range(n_stages):
        loads.pop(i).wait()
        if i >= 2:
            stores.pop(i - 2).wait()
        inb = inbufs[i % 2]
        outb = outbufs[i % 2]

        @plsc.parallel_loop(0, _CHUNK, _L, unroll=8)
        def _(v, inb=inb, outb=outb):
            x = inb[pl.ds(v, _L)]
            q = (x.astype(jnp.float32) * _INV).astype(jnp.int32)
            r = x - q * _M
            r = jnp.where(r < 0, r + _M, r)
            r = jnp.where(r >= _M, r - _M, r)
            outb[pl.ds(v, _L)] = r

        stores[i] = start_store(i)
        if i + 2 < n_stages:
            loads[i + 2] = start_load(i + 2)
    stores.pop(n_stages - 2).wait()
    stores.pop(n_stages - 1).wait()


_sc_mod2 = functools.partial(
    pl.kernel,
    out_type=(jax.ShapeDtypeStruct((_N,), jnp.int32),
              jax.ShapeDtypeStruct((_N,), jnp.int32)),
    mesh=plsc.VectorSubcoreMesh(core_axis_name="c", subcore_axis_name="s"),
    scratch_types=(
        pltpu.VMEM((_CHUNK,), jnp.int32),
        pltpu.VMEM((_CHUNK,), jnp.int32),
        pltpu.VMEM((_CHUNK,), jnp.int32),
        pltpu.VMEM((_CHUNK,), jnp.int32),
        pltpu.SemaphoreType.DMA,
        pltpu.SemaphoreType.DMA,
        pltpu.SemaphoreType.DMA,
        pltpu.SemaphoreType.DMA,
    ),
)(_mod_body)


def kernel(id_list_offsets, id_list_values, id_score_list_offsets,
           id_score_list_keys, id_score_list_scores):
    idx_values, idx_keys = _sc_mod2(id_list_values, id_score_list_keys)
    return (id_list_offsets, idx_values, id_score_list_offsets, idx_keys,
            id_score_list_scores.astype(jnp.float32))


# speedup vs baseline: 1.0189x; 1.0189x over previous
import functools

import jax
import jax.numpy as jnp
from jax import lax
from jax.experimental import pallas as pl
from jax.experimental.pallas import tpu as pltpu
from jax.experimental.pallas import tpu_sc as plsc

_M = 1000000
_C = float((1.0 / _M) * (1.0 - 2.0**-22))

_NC, _NS, _L = 2, 16, 16
_NW = _NC * _NS
_N = 3276800
_PER_W = _N // _NW
_CHUNK = 12800
_NCHUNK = _PER_W // _CHUNK


def _mod_vec(x):
    q = (x.astype(jnp.float32) * jnp.float32(_C)).astype(jnp.int32)
    r = x - q * _M
    ru = lax.bitcast_convert_type(r, jnp.uint32)
    su = lax.bitcast_convert_type(r - _M, jnp.uint32)
    return lax.bitcast_convert_type(jnp.minimum(ru, su), jnp.int32)


def _mod_body(src_hbm, dst_hbm, in0, in1, ou0, ou1, ls0, ls1, ss0, ss1):
    wid = lax.axis_index("s") * _NC + lax.axis_index("c")
    base = wid * _PER_W

    inbufs = (in0, in1)
    outbufs = (ou0, ou1)
    lsems = (ls0, ls1)
    ssems = (ss0, ss1)

    def start_load(i):
        return pltpu.async_copy(
            src_hbm.at[pl.ds(base + i * _CHUNK, _CHUNK)],
            inbufs[i % 2], lsems[i % 2])

    def start_store(i):
        return pltpu.async_copy(
            outbufs[i % 2],
            dst_hbm.at[pl.ds(base + i * _CHUNK, _CHUNK)], ssems[i % 2])

    loads = {0: start_load(0), 1: start_load(1)}
    stores = {}
    for i in range(_NCHUNK):
        loads.pop(i).wait()
        if i >= 2:
            stores.pop(i - 2).wait()
        inb = inbufs[i % 2]
        outb = outbufs[i % 2]

        @plsc.parallel_loop(0, _CHUNK, _L, unroll=8)
        def _(v, inb=inb, outb=outb):
            outb[pl.ds(v, _L)] = _mod_vec(inb[pl.ds(v, _L)])

        stores[i] = start_store(i)
        if i + 2 < _NCHUNK:
            loads[i + 2] = start_load(i + 2)
    stores.pop(_NCHUNK - 2).wait()
    stores.pop(_NCHUNK - 1).wait()


_sc_mod = functools.partial(
    pl.kernel,
    out_type=jax.ShapeDtypeStruct((_N,), jnp.int32),
    mesh=plsc.VectorSubcoreMesh(core_axis_name="c", subcore_axis_name="s"),
    scratch_types=(
        pltpu.VMEM((_CHUNK,), jnp.int32),
        pltpu.VMEM((_CHUNK,), jnp.int32),
        pltpu.VMEM((_CHUNK,), jnp.int32),
        pltpu.VMEM((_CHUNK,), jnp.int32),
        pltpu.SemaphoreType.DMA,
        pltpu.SemaphoreType.DMA,
        pltpu.SemaphoreType.DMA,
        pltpu.SemaphoreType.DMA,
    ),
)(_mod_body)


def kernel(id_list_offsets, id_list_values, id_score_list_offsets,
           id_score_list_keys, id_score_list_scores):
    idx_values = _sc_mod(id_list_values)
    idx_keys = _sc_mod(id_score_list_keys)
    return (id_list_offsets, idx_values, id_score_list_offsets, idx_keys,
            id_score_list_scores.astype(jnp.float32))
